# baseline (speedup 1.0000x reference)
import functools

import jax
import jax.numpy as jnp
from jax import lax
from jax.experimental import pallas as pl
from jax.experimental.pallas import tpu as pltpu
from jax.experimental.pallas import tpu_sc as plsc

NC = 2
NS = 16
NW = NC * NS
L = 16
CB = 1024


def _ceil_to(x, m):
    return (x + m - 1) // m * m


@functools.lru_cache(maxsize=None)
def _make_sc_call(T, N, E):
    assert T == L, "kernel assumes one timestep per vector lane"
    EW = _ceil_to(E, NW * 2 * CB) // NW
    NCHUNK = EW // CB
    NCH2 = NCHUNK // 2
    EP = EW * NW
    NP = _ceil_to(N, NW * 8)
    RW = NP // NW

    mesh = plsc.VectorSubcoreMesh(core_axis_name="c", subcore_axis_name="s")

    def body(tabu, tabv, ei3, outpe, outke,
             idx0, idx1, gr0, gc0, gr1, gc1, vbuf, osc,
             semi0, semi1, semg0, semg1, semv):
        wid = lax.axis_index("s") * NC + lax.axis_index("c")
        cbase = wid * NCHUNK
        zero = jnp.zeros((L,), jnp.float32)

        def idx_start(c, idx, semi):
            pltpu.async_copy(ei3.at[cbase + c], idx, semi)

        def idx_wait(idx, semi):
            pltpu.make_async_copy(ei3.at[cbase], idx, semi).wait()

        def g_start(idx, gr, gc, semg):
            pltpu.async_copy(tabu.at[idx.at[0]], gr, semg)
            pltpu.async_copy(tabu.at[idx.at[1]], gc, semg)

        def g_wait(idx, gr, gc, semg):
            pltpu.make_async_copy(tabu.at[idx.at[0]], gr, semg).wait()
            pltpu.make_async_copy(tabu.at[idx.at[1]], gc, semg).wait()

        def accum(gr, gc, acc):
            def body8(j, accs):
                a0, a1, a2, a3 = accs
                e = j * 8
                d = gr[e] - gc[e]
                a0 = a0 + d * d
                d = gr[e + 1] - gc[e + 1]
                a1 = a1 + d * d
                d = gr[e + 2] - gc[e + 2]
                a2 = a2 + d * d
                d = gr[e + 3] - gc[e + 3]
                a3 = a3 + d * d
                d = gr[e + 4] - gc[e + 4]
                a0 = a0 + d * d
                d = gr[e + 5] - gc[e + 5]
                a1 = a1 + d * d
                d = gr[e + 6] - gc[e + 6]
                a2 = a2 + d * d
                d = gr[e + 7] - gc[e + 7]
                a3 = a3 + d * d
                return (a0, a1, a2, a3)

            accs = lax.fori_loop(0, CB // 8, body8, (zero, zero, zero, zero),
                                 unroll=2)
            return acc + (accs[0] + accs[1]) + (accs[2] + accs[3])

        pltpu.async_copy(tabv.at[pl.ds(wid * RW, RW)], vbuf, semv)
        idx_start(0, idx0, semi0)
        idx_start(1, idx1, semi1)
        idx_wait(idx0, semi0)
        g_start(idx0, gr0, gc0, semg0)
        idx_wait(idx1, semi1)
        g_start(idx1, gr1, gc1, semg1)

        def chunk2(k, acc):
            g_wait(idx0, gr0, gc0, semg0)

            @pl.when(k < NCH2 - 1)
            def _():
                idx_start(2 * k + 2, idx0, semi0)

            acc = accum(gr0, gc0, acc)

            @pl.when(k < NCH2 - 1)
            def _():
                idx_wait(idx0, semi0)
                g_start(idx0, gr0, gc0, semg0)

            g_wait(idx1, gr1, gc1, semg1)

            @pl.when(k < NCH2 - 1)
            def _():
                idx_start(2 * k + 3, idx1, semi1)

            acc = accum(gr1, gc1, acc)

            @pl.when(k < NCH2 - 1)
            def _():
                idx_wait(idx1, semi1)
                g_start(idx1, gr1, gc1, semg1)

            return acc

        pe = lax.fori_loop(0, NCH2, chunk2, zero)
        osc[...] = pe
        pltpu.sync_copy(osc, outpe.at[wid])

        pltpu.make_async_copy(tabv.at[pl.ds(wid * RW, RW)], vbuf, semv).wait()

        def krow8(j, accs):
            a0, a1, a2, a3 = accs
            r = j * 8
            x = vbuf[r]
            a0 = a0 + x * x
            x = vbuf[r + 1]
            a1 = a1 + x * x
            x = vbuf[r + 2]
            a2 = a2 + x * x
            x = vbuf[r + 3]
            a3 = a3 + x * x
            x = vbuf[r + 4]
            a0 = a0 + x * x
            x = vbuf[r + 5]
            a1 = a1 + x * x
            x = vbuf[r + 6]
            a2 = a2 + x * x
            x = vbuf[r + 7]
            a3 = a3 + x * x
            return (a0, a1, a2, a3)

        ka = lax.fori_loop(0, RW // 8, krow8, (zero, zero, zero, zero),
                           unroll=2)
        osc[...] = (ka[0] + ka[1]) + (ka[2] + ka[3])
        pltpu.sync_copy(osc, outke.at[wid])

    call = pl.kernel(
        body,
        out_type=(
            jax.ShapeDtypeStruct((NW, L), jnp.float32),
            jax.ShapeDtypeStruct((NW, L), jnp.float32),
        ),
        mesh=mesh,
        scratch_types=[
            pltpu.VMEM((2, CB), jnp.int32),
            pltpu.VMEM((2, CB), jnp.int32),
            pltpu.VMEM((CB, L), jnp.float32),
            pltpu.VMEM((CB, L), jnp.float32),
            pltpu.VMEM((CB, L), jnp.float32),
            pltpu.VMEM((CB, L), jnp.float32),
            pltpu.VMEM((RW, L), jnp.float32),
            pltpu.VMEM((L,), jnp.float32),
            pltpu.SemaphoreType.DMA,
            pltpu.SemaphoreType.DMA,
            pltpu.SemaphoreType.DMA,
            pltpu.SemaphoreType.DMA,
            pltpu.SemaphoreType.DMA,
        ],
        compiler_params=pltpu.CompilerParams(use_tc_tiling_on_sc=False),
    )
    return call, EP, NP


def kernel(states, edge_index):
    T, N, _ = states.shape
    E = edge_index.shape[1]
    call, EP, NP = _make_sc_call(T, N, E)

    tabu = states[:, :, 0].T
    tabv = jnp.pad(states[:, :, 1].T, ((0, NP - N), (0, 0)))
    ei = edge_index.astype(jnp.int32)
    eip = jnp.pad(ei, ((0, 0), (0, EP - E)))
    ei3 = eip.reshape(2, EP // CB, CB).transpose(1, 0, 2)
    outpe, outke = call(tabu, tabv, ei3)

    H = 0.5 * (jnp.sum(outpe, axis=0) + jnp.sum(outke, axis=0))
    dH = H[1:] - H[:-1]
    return jnp.sum(dH * dH) / (T - 1)
